# baseline (speedup 1.0000x reference)
import jax
import jax.numpy as jnp
from jax import lax
from jax.experimental import pallas as pl
from jax.experimental.pallas import tpu as pltpu
from jax.experimental.pallas import tpu_sc as plsc

_NC = 2
_NS = 16
_NW = _NC * _NS
_L = 16

_N = 16384
_F = 2
_ROWS_PER = _N // _NW
_VECS = _ROWS_PER * _F // _L


def _body(x_hbm, labels_hbm, cm_hbm, out_hbm, x_v, lab_v, cm_v, out_v):
    wid = lax.axis_index("s") * _NC + lax.axis_index("c")
    rbase = wid * _ROWS_PER

    pltpu.sync_copy(x_hbm.at[pl.ds(rbase, _ROWS_PER)], x_v)
    pltpu.sync_copy(labels_hbm.at[pl.ds(rbase, _ROWS_PER)], lab_v)
    pltpu.sync_copy(cm_hbm, cm_v)

    iota = lax.iota(jnp.int32, _L)
    half = iota >> 1
    feat = iota & 1

    for i in range(_VECS):
        row = i * (_L // _F) + half
        lab = plsc.load_gather(lab_v, [row])
        g = plsc.load_gather(cm_v, [lab, feat])
        xv = plsc.load_gather(x_v, [row, feat])
        plsc.store_scatter(out_v, [row, feat], xv - g)

    pltpu.sync_copy(out_v, out_hbm.at[pl.ds(rbase, _ROWS_PER)])


_sc_call = pl.kernel(
    _body,
    out_type=jax.ShapeDtypeStruct((_N, _F), jnp.float32),
    mesh=plsc.VectorSubcoreMesh(core_axis_name="c", subcore_axis_name="s"),
    compiler_params=pltpu.CompilerParams(
        needs_layout_passes=False, use_tc_tiling_on_sc=False
    ),
    scratch_types=[
        pltpu.VMEM((_ROWS_PER, _F), jnp.float32),
        pltpu.VMEM((_ROWS_PER,), jnp.int32),
        pltpu.VMEM((3, _F), jnp.float32),
        pltpu.VMEM((_ROWS_PER, _F), jnp.float32),
    ],
)


@jax.jit
def kernel(x, labels, class_means, global_mean):
    del global_mean
    return _sc_call(x, labels.astype(jnp.int32), class_means)


# speedup vs baseline: 1.3476x; 1.3476x over previous
import jax
import jax.numpy as jnp
from jax import lax
from jax.experimental import pallas as pl
from jax.experimental.pallas import tpu as pltpu
from jax.experimental.pallas import tpu_sc as plsc

_NC = 2
_NS = 16
_NW = _NC * _NS
_L = 16

_N = 16384
_F = 2
_ROWS_PER = _N // _NW
_VECS = _ROWS_PER * _F // _L


def _body(x_hbm, labels_hbm, cm_hbm, out_hbm, x_v, lab_v, cm_v):
    wid = lax.axis_index("s") * _NC + lax.axis_index("c")
    rbase = wid * _ROWS_PER

    pltpu.sync_copy(x_hbm.at[pl.ds(rbase, _ROWS_PER)], x_v)
    pltpu.sync_copy(labels_hbm.at[pl.ds(rbase, _ROWS_PER)], lab_v)
    pltpu.sync_copy(cm_hbm, cm_v)

    iota = lax.iota(jnp.int32, _L)
    half = iota >> 1
    feat = iota & 1

    for i in range(_VECS):
        row = i * (_L // _F) + half
        lab = plsc.load_gather(lab_v, [row])
        g = plsc.load_gather(cm_v, [lab, feat])
        xv = plsc.load_gather(x_v, [row, feat])
        plsc.store_scatter(x_v, [row, feat], xv - g)

    pltpu.sync_copy(x_v, out_hbm.at[pl.ds(rbase, _ROWS_PER)])


_sc_call = pl.kernel(
    _body,
    out_type=jax.ShapeDtypeStruct((_N, _F), jnp.float32),
    mesh=plsc.VectorSubcoreMesh(core_axis_name="c", subcore_axis_name="s"),
    compiler_params=pltpu.CompilerParams(needs_layout_passes=False),
    scratch_types=[
        pltpu.VMEM((_ROWS_PER, _F), jnp.float32),
        pltpu.VMEM((_ROWS_PER,), jnp.int32),
        pltpu.VMEM((3, _F), jnp.float32),
    ],
)


@jax.jit
def kernel(x, labels, class_means, global_mean):
    del global_mean
    return _sc_call(x, labels.astype(jnp.int32), class_means)


# speedup vs baseline: 2.4258x; 1.8001x over previous
import jax
import jax.numpy as jnp
from jax import lax
from jax.experimental import pallas as pl
from jax.experimental.pallas import tpu as pltpu
from jax.experimental.pallas import tpu_sc as plsc

_NC = 2
_NS = 16
_NW = _NC * _NS
_L = 16

_N = 16384
_F = 2
_B = 128
_NB = _N // _B
_BLKS_PER = _NB // _NW
_ROWS_PER = _BLKS_PER * _B
_CHUNKS = _B // _L


def _body(x_hbm, labels_hbm, cm_hbm, out_hbm, x_v, lab_v, cm_v):
    wid = lax.axis_index("s") * _NC + lax.axis_index("c")
    bbase = wid * _BLKS_PER
    rbase = wid * _ROWS_PER

    pltpu.sync_copy(x_hbm.at[pl.ds(bbase, _BLKS_PER)], x_v)
    pltpu.sync_copy(labels_hbm.at[pl.ds(rbase, _ROWS_PER)], lab_v)
    pltpu.sync_copy(cm_hbm, cm_v)

    for b in range(_BLKS_PER):
        for f in range(_F):
            fvec = jnp.full((_L,), f, jnp.int32)
            for c in range(_CHUNKS):
                rlo = b * _B + c * _L
                lab = lab_v[pl.ds(rlo, _L)]
                g = plsc.load_gather(cm_v, [lab, fvec])
                x_v[b, f, pl.ds(c * _L, _L)] -= g

    pltpu.sync_copy(x_v, out_hbm.at[pl.ds(bbase, _BLKS_PER)])


_sc_call = pl.kernel(
    _body,
    out_type=jax.ShapeDtypeStruct((_NB, _F, _B), jnp.float32),
    mesh=plsc.VectorSubcoreMesh(core_axis_name="c", subcore_axis_name="s"),
    compiler_params=pltpu.CompilerParams(
        needs_layout_passes=False, use_tc_tiling_on_sc=False
    ),
    scratch_types=[
        pltpu.VMEM((_BLKS_PER, _F, _B), jnp.float32),
        pltpu.VMEM((_ROWS_PER,), jnp.int32),
        pltpu.VMEM((3, _F), jnp.float32),
    ],
)


@jax.jit
def kernel(x, labels, class_means, global_mean):
    del global_mean
    x3 = jnp.transpose(x.reshape(_NB, _B, _F), (0, 2, 1))
    out3 = _sc_call(x3, labels.astype(jnp.int32), class_means)
    return jnp.transpose(out3, (0, 2, 1)).reshape(_N, _F)


# speedup vs baseline: 2.5294x; 1.0427x over previous
import jax
import jax.numpy as jnp
from jax import lax
from jax.experimental import pallas as pl
from jax.experimental.pallas import tpu as pltpu
from jax.experimental.pallas import tpu_sc as plsc

_NC = 2
_NS = 16
_NW = _NC * _NS
_L = 16

_N = 16384
_F = 2
_B = 128
_NB = _N // _B
_BLKS_PER = _NB // _NW
_ROWS_PER = _BLKS_PER * _B
_CHUNKS = _B // _L


def _body(x_hbm, labels_hbm, cm_hbm, out_hbm, x_v, lab_v, cm_v, sem):
    wid = lax.axis_index("s") * _NC + lax.axis_index("c")
    bbase = wid * _BLKS_PER
    rbase = wid * _ROWS_PER

    cx = pltpu.async_copy(x_hbm.at[pl.ds(bbase, _BLKS_PER)], x_v, sem)
    cl = pltpu.async_copy(labels_hbm.at[pl.ds(rbase, _ROWS_PER)], lab_v, sem)
    cc = pltpu.async_copy(cm_hbm, cm_v, sem)
    cx.wait()
    cl.wait()
    cc.wait()

    for b in range(_BLKS_PER):
        for f in range(_F):
            fvec = jnp.full((_L,), f, jnp.int32)
            for c in range(_CHUNKS):
                rlo = b * _B + c * _L
                lab = lab_v[pl.ds(rlo, _L)]
                g = plsc.load_gather(cm_v, [lab, fvec])
                x_v[b, f, pl.ds(c * _L, _L)] -= g

    pltpu.sync_copy(x_v, out_hbm.at[pl.ds(bbase, _BLKS_PER)])


_sc_call = pl.kernel(
    _body,
    out_type=jax.ShapeDtypeStruct((_NB, _F, _B), jnp.float32),
    mesh=plsc.VectorSubcoreMesh(core_axis_name="c", subcore_axis_name="s"),
    compiler_params=pltpu.CompilerParams(
        needs_layout_passes=False, use_tc_tiling_on_sc=False
    ),
    scratch_types=[
        pltpu.VMEM((_BLKS_PER, _F, _B), jnp.float32),
        pltpu.VMEM((_ROWS_PER,), jnp.int32),
        pltpu.VMEM((3, _F), jnp.float32),
        pltpu.SemaphoreType.DMA,
    ],
)


@jax.jit
def kernel(x, labels, class_means, global_mean):
    del global_mean
    x3 = jnp.transpose(x.reshape(_NB, _B, _F), (0, 2, 1))
    out3 = _sc_call(x3, labels.astype(jnp.int32), class_means)
    return jnp.transpose(out3, (0, 2, 1)).reshape(_N, _F)
